# baseline (speedup 1.0000x reference)
import functools

import jax
import jax.numpy as jnp
from jax import lax
from jax.experimental import pallas as pl
from jax.experimental.pallas import tpu as pltpu
from jax.experimental.pallas import tpu_sc as plsc

NC = 2
NS = 16
NW = NC * NS
D = 64
SEQ = 200
SPC = 4
CHUNK = SPC * SEQ
M = 2


def _make_lookup(n_seq):
    assert (n_seq * SEQ) % (NW * CHUNK) == 0
    b_per_w = n_seq * SEQ // NW
    n_steps = b_per_w // CHUNK
    n_groups = n_steps // M
    assert n_groups >= 3 and n_steps % M == 0
    mesh = plsc.VectorSubcoreMesh(
        core_axis_name="c", subcore_axis_name="s",
        num_cores=NC, num_subcores=NS)

    @functools.partial(
        pl.kernel,
        mesh=mesh,
        compiler_params=pltpu.CompilerParams(use_tc_tiling_on_sc=False),
        out_type=jax.ShapeDtypeStruct((n_seq, SEQ, D), jnp.float32),
        scratch_types=[
            pltpu.VMEM((M, CHUNK), jnp.int32),
            pltpu.VMEM((M, CHUNK, D), jnp.float32),
            pltpu.SemaphoreType.DMA((M,)),
            pltpu.SemaphoreType.DMA((M,)),
            pltpu.SemaphoreType.DMA((M,)),
        ],
    )
    def lookup(table_hbm, idx_hbm, out_hbm, idx_v, rows_v, sem_i, sem_g, sem_o):
        wid = lax.axis_index("s") * NC + lax.axis_index("c")
        base = wid * b_per_w
        seq_base = wid * (b_per_w // SEQ)

        def load(step, slot):
            pltpu.async_copy(
                idx_hbm.at[pl.ds(base + step * CHUNK, CHUNK)],
                idx_v.at[slot], sem_i.at[slot])

        def gather(step, slot):
            pltpu.async_copy(
                table_hbm.at[idx_v.at[slot]], rows_v.at[slot], sem_g.at[slot])

        def store(step, slot):
            s0 = seq_base + step * SPC
            for k in range(SPC):
                pltpu.async_copy(
                    rows_v.at[slot, pl.ds(k * SEQ, SEQ)],
                    out_hbm.at[s0 + k], sem_o.at[slot])

        def wait_rows(sem, slot):
            pltpu.make_async_copy(table_hbm.at[pl.ds(0, CHUNK)],
                                  rows_v.at[slot], sem.at[slot]).wait()

        def wait_idx(slot):
            pltpu.make_async_copy(idx_hbm.at[pl.ds(0, CHUNK)],
                                  idx_v.at[slot], sem_i.at[slot]).wait()

        for b in range(M - 1):
            load(b, b)
        for b in range(M):
            i = b
            if i > 0:
                p = (b - 1) % M
                wait_rows(sem_g, p)
                store(i - 1, p)
                load(i + M - 1, p)
            else:
                load(M - 1, (M - 1) % M)
            wait_idx(b)
            gather(i, b)

        def group_body(g, carry):
            i0 = g * M
            for b in range(M):
                i = i0 + b
                p = (b - 1) % M
                wait_rows(sem_g, p)
                store(i - 1, p)
                load(i + M - 1, p)
                wait_idx(b)
                wait_rows(sem_o, b)
                gather(i, b)
            return carry

        lax.fori_loop(1, n_groups - 1, group_body, 0)

        i0 = (n_groups - 1) * M
        for b in range(M):
            i = i0 + b
            p = (b - 1) % M
            wait_rows(sem_g, p)
            store(i - 1, p)
            if i + M - 1 < n_steps:
                load(i + M - 1, p)
            wait_idx(b)
            wait_rows(sem_o, b)
            gather(i, b)

        wait_rows(sem_g, (n_steps - 1) % M)
        store(n_steps - 1, (n_steps - 1) % M)
        for b in range(M):
            wait_rows(sem_o, b)

    return lookup


NSPLIT = 4


@jax.jit
def _impl(pos_ids, table):
    n_seq = pos_ids.shape[0]
    part = n_seq // NSPLIT
    lookup = _make_lookup(part)
    out = jnp.zeros((n_seq, SEQ, D), jnp.float32)
    for p in range(NSPLIT):
        flat = pos_ids[p * part:(p + 1) * part].reshape(-1).astype(jnp.int32)
        out = lax.dynamic_update_slice(out, lookup(table, flat),
                                       (p * part, 0, 0))
    return out


def kernel(pos_ids, position_encoding):
    return _impl(pos_ids, position_encoding)


# speedup vs baseline: 1.4762x; 1.4762x over previous
import functools

import jax
import jax.numpy as jnp
from jax import lax
from jax.experimental import pallas as pl
from jax.experimental.pallas import tpu as pltpu
from jax.experimental.pallas import tpu_sc as plsc

NC = 2
NS = 16
NW = NC * NS
D = 64
SEQ = 200
SPC = 2
CHUNK = SPC * SEQ
M = 4
LAG = 2


def _make_lookup(n_seq):
    assert (n_seq * SEQ) % (NW * CHUNK) == 0
    b_per_w = n_seq * SEQ // NW
    n_steps = b_per_w // CHUNK
    n_groups = n_steps // M
    assert n_groups >= 3 and n_steps % M == 0
    mesh = plsc.VectorSubcoreMesh(
        core_axis_name="c", subcore_axis_name="s",
        num_cores=NC, num_subcores=NS)

    @functools.partial(
        pl.kernel,
        mesh=mesh,
        compiler_params=pltpu.CompilerParams(use_tc_tiling_on_sc=False),
        out_type=jax.ShapeDtypeStruct((n_seq, SEQ, D), jnp.float32),
        scratch_types=[
            pltpu.VMEM((M, CHUNK), jnp.int32),
            pltpu.VMEM((M, CHUNK, D), jnp.float32),
            pltpu.SemaphoreType.DMA((M,)),
            pltpu.SemaphoreType.DMA((M,)),
            pltpu.SemaphoreType.DMA((M,)),
        ],
    )
    def lookup(table_hbm, idx_hbm, out_hbm, idx_v, rows_v, sem_i, sem_g, sem_o):
        wid = lax.axis_index("s") * NC + lax.axis_index("c")
        base = wid * b_per_w
        seq_base = wid * (b_per_w // SEQ)

        def load(step, slot):
            pltpu.async_copy(
                idx_hbm.at[pl.ds(base + step * CHUNK, CHUNK)],
                idx_v.at[slot], sem_i.at[slot])

        def gather(step, slot):
            pltpu.async_copy(
                table_hbm.at[idx_v.at[slot]], rows_v.at[slot], sem_g.at[slot])

        def store(step, slot):
            s0 = seq_base + step * SPC
            for k in range(SPC):
                pltpu.async_copy(
                    rows_v.at[slot, pl.ds(k * SEQ, SEQ)],
                    out_hbm.at[s0 + k], sem_o.at[slot])

        def wait_rows(sem, slot):
            pltpu.make_async_copy(table_hbm.at[pl.ds(0, CHUNK)],
                                  rows_v.at[slot], sem.at[slot]).wait()

        def wait_idx(slot):
            pltpu.make_async_copy(idx_hbm.at[pl.ds(0, CHUNK)],
                                  idx_v.at[slot], sem_i.at[slot]).wait()

        for b in range(LAG):
            load(b, b)
        for b in range(M):
            i = b
            if i >= LAG:
                p = (b - LAG) % M
                wait_rows(sem_g, p)
                store(i - LAG, p)
                load(i - LAG + M, p)
            else:
                load(i + LAG, (i + LAG) % M)
            wait_idx(b)
            gather(i, b)

        def group_body(g, carry):
            i0 = g * M
            for b in range(M):
                i = i0 + b
                p = (b - LAG) % M
                wait_rows(sem_g, p)
                store(i - LAG, p)
                load(i - LAG + M, p)
                wait_idx(b)
                wait_rows(sem_o, b)
                gather(i, b)
            return carry

        lax.fori_loop(1, n_groups - 1, group_body, 0)

        i0 = (n_groups - 1) * M
        for b in range(M):
            i = i0 + b
            p = (b - LAG) % M
            wait_rows(sem_g, p)
            store(i - LAG, p)
            if i - LAG + M < n_steps:
                load(i - LAG + M, p)
            wait_idx(b)
            wait_rows(sem_o, b)
            gather(i, b)

        for i in range(n_steps - LAG, n_steps):
            wait_rows(sem_g, i % M)
            store(i, i % M)
        for b in range(M):
            wait_rows(sem_o, b)

    return lookup


@jax.jit
def _impl(pos_ids, table):
    flat = pos_ids.reshape(-1).astype(jnp.int32)
    return _make_lookup(pos_ids.shape[0])(table, flat)


def kernel(pos_ids, position_encoding):
    return _impl(pos_ids, position_encoding)
